# baseline (speedup 1.0000x reference)
import jax
import jax.numpy as jnp
from jax import lax
from jax.experimental import pallas as pl
from jax.experimental.pallas import tpu as pltpu
from jax.experimental.pallas import tpu_sc as plsc

ROWS = 2097152
COLS = 4
LANE = 128
IN_R = ROWS * COLS // LANE
OUT_R = 2 * IN_R
CHUNK_R = 8192
NC = 2
NS = 16
NW = NC * NS
S = IN_R // NW


def _body(in_hbm, zeros_hbm, out_hbm, sem_a, sem_b):
    c = lax.axis_index("c")
    s = lax.axis_index("s")
    wid = s * NC + c
    in_off = wid * S
    k = wid // 4
    q = wid % 4
    out_off = k * (2 * CHUNK_R) + q * S
    zero_off = out_off + CHUNK_R
    cp = pltpu.async_copy(
        in_hbm.at[pl.ds(in_off, S)], out_hbm.at[pl.ds(out_off, S)], sem_a)
    zp = pltpu.async_copy(
        zeros_hbm.at[:], out_hbm.at[pl.ds(zero_off, S)], sem_b)
    cp.wait()
    zp.wait()


def kernel(psi):
    flat = psi.reshape(ROWS // LANE, LANE, COLS).transpose(0, 2, 1)
    flat = flat.reshape(IN_R, LANE)
    zeros = jnp.zeros((S, LANE), jnp.float32)
    mesh = plsc.VectorSubcoreMesh(core_axis_name="c", subcore_axis_name="s")
    run = pl.kernel(
        _body,
        out_type=jax.ShapeDtypeStruct((OUT_R, LANE), jnp.float32),
        mesh=mesh,
        scratch_types=[pltpu.SemaphoreType.DMA, pltpu.SemaphoreType.DMA],
    )
    out = run(flat, zeros)
    out = out.reshape(2 * ROWS // LANE, COLS, LANE).transpose(0, 2, 1)
    return out.reshape(2 * ROWS, COLS)


# speedup vs baseline: 31.2026x; 31.2026x over previous
import jax
import jax.numpy as jnp
from jax import lax
from jax.experimental import pallas as pl
from jax.experimental.pallas import tpu as pltpu
from jax.experimental.pallas import tpu_sc as plsc

ROWS = 2097152
COLS = 4
LANE = 128
IN_R = ROWS * COLS // LANE
OUT_R = 2 * IN_R
CHUNK_R = 8192
NC = 2
NS = 16
NW = NC * NS
S = IN_R // NW
BR = 128
NCH = S // BR
NBUF = 4


def _body(in_hbm, zeros_hbm, out_hbm,
          b0, b1, b2, b3, zbuf,
          si0, si1, si2, si3, so0, so1, so2, so3, sz, szi):
    bufs = (b0, b1, b2, b3)
    sin = (si0, si1, si2, si3)
    sout = (so0, so1, so2, so3)
    c = lax.axis_index("c")
    s = lax.axis_index("s")
    wid = s * NC + c
    in_off = wid * S
    k = wid // 4
    q = wid % 4
    out_off = k * (2 * CHUNK_R) + q * S
    zero_off = out_off + CHUNK_R

    pltpu.async_copy(zeros_hbm.at[:], zbuf, szi).wait()
    zh = [pltpu.async_copy(
        zbuf, out_hbm.at[pl.ds(zero_off + j * BR, BR)], sz)
        for j in range(NCH)]

    in_h = [None] * NCH
    out_h = [None] * NCH
    for j in range(NCH):
        b = j % NBUF
        if j >= NBUF:
            out_h[j - NBUF].wait()
        in_h[j] = pltpu.async_copy(
            in_hbm.at[pl.ds(in_off + j * BR, BR)], bufs[b], sin[b])
        in_h[j].wait()
        out_h[j] = pltpu.async_copy(
            bufs[b], out_hbm.at[pl.ds(out_off + j * BR, BR)], sout[b])
    for j in range(NCH - NBUF, NCH):
        out_h[j].wait()
    for h in zh:
        h.wait()


def kernel(psi):
    flat = psi.reshape(ROWS // LANE, LANE, COLS).transpose(0, 2, 1)
    flat = flat.reshape(IN_R, LANE)
    zeros = jnp.zeros((BR, LANE), jnp.float32)
    mesh = plsc.VectorSubcoreMesh(core_axis_name="c", subcore_axis_name="s")
    run = pl.kernel(
        _body,
        out_type=jax.ShapeDtypeStruct((OUT_R, LANE), jnp.float32),
        mesh=mesh,
        scratch_types=(
            [pltpu.VMEM((BR, LANE), jnp.float32)] * (NBUF + 1)
            + [pltpu.SemaphoreType.DMA] * 10
        ),
    )
    out = run(flat, zeros)
    out = out.reshape(2 * ROWS // LANE, COLS, LANE).transpose(0, 2, 1)
    return out.reshape(2 * ROWS, COLS)


# speedup vs baseline: 34.3680x; 1.1014x over previous
import jax
import jax.numpy as jnp
from jax import lax
from jax.experimental import pallas as pl
from jax.experimental.pallas import tpu as pltpu
from jax.experimental.pallas import tpu_sc as plsc

ROWS = 2097152
COLS = 4
LANE = 128
IN_R = ROWS * COLS // LANE
OUT_R = 2 * IN_R
CHUNK_R = 8192
NC = 2
NS = 16
NW = NC * NS
S = IN_R // NW
BR = 256
NCH = S // BR
NBUF = 2


def _body(in_hbm, zeros_hbm, out_hbm,
          b0, b1, zbuf,
          si0, si1, so0, so1, sz, szi):
    bufs = (b0, b1)
    sin = (si0, si1)
    sout = (so0, so1)
    c = lax.axis_index("c")
    s = lax.axis_index("s")
    wid = s * NC + c
    in_off = wid * S
    k = wid // 4
    q = wid % 4
    out_off = k * (2 * CHUNK_R) + q * S
    zero_off = out_off + CHUNK_R

    pltpu.async_copy(zeros_hbm.at[:], zbuf, szi).wait()
    zh = [pltpu.async_copy(
        zbuf, out_hbm.at[pl.ds(zero_off + j * BR, BR)], sz)
        for j in range(NCH)]

    in_h = [None] * NCH
    out_h = [None] * NCH
    for j in range(NCH):
        b = j % NBUF
        if j >= NBUF:
            out_h[j - NBUF].wait()
        in_h[j] = pltpu.async_copy(
            in_hbm.at[pl.ds(in_off + j * BR, BR)], bufs[b], sin[b])
        in_h[j].wait()
        out_h[j] = pltpu.async_copy(
            bufs[b], out_hbm.at[pl.ds(out_off + j * BR, BR)], sout[b])
    for j in range(NCH - NBUF, NCH):
        out_h[j].wait()
    for h in zh:
        h.wait()


def kernel(psi):
    flat = psi.reshape(ROWS // LANE, LANE, COLS).transpose(0, 2, 1)
    flat = flat.reshape(IN_R, LANE)
    zeros = jnp.zeros((BR, LANE), jnp.float32)
    mesh = plsc.VectorSubcoreMesh(core_axis_name="c", subcore_axis_name="s")
    run = pl.kernel(
        _body,
        out_type=jax.ShapeDtypeStruct((OUT_R, LANE), jnp.float32),
        mesh=mesh,
        scratch_types=(
            [pltpu.VMEM((BR, LANE), jnp.float32)] * (NBUF + 1)
            + [pltpu.SemaphoreType.DMA] * 6
        ),
    )
    out = run(flat, zeros)
    out = out.reshape(2 * ROWS // LANE, COLS, LANE).transpose(0, 2, 1)
    return out.reshape(2 * ROWS, COLS)


# speedup vs baseline: 35.5557x; 1.0346x over previous
import jax
import jax.numpy as jnp
from jax import lax
from jax.experimental import pallas as pl
from jax.experimental.pallas import tpu as pltpu
from jax.experimental.pallas import tpu_sc as plsc

ROWS = 2097152
COLS = 4
LANE = 128
IN_R = ROWS * COLS // LANE
OUT_R = 2 * IN_R
CHUNK_R = 8192
NC = 2
NS = 16
NW = NC * NS
S = IN_R // NW
BR = 256
NCH = S // BR
NBUF = 3
ZBR = 128
ZCH = S // ZBR


def _body(in_hbm, zeros_hbm, out_hbm,
          b0, b1, b2, zbuf,
          si0, si1, si2, so0, so1, so2, sz, szi):
    bufs = (b0, b1, b2)
    sin = (si0, si1, si2)
    sout = (so0, so1, so2)
    c = lax.axis_index("c")
    s = lax.axis_index("s")
    wid = s * NC + c
    in_off = wid * S
    k = wid // 4
    q = wid % 4
    out_off = k * (2 * CHUNK_R) + q * S
    zero_off = out_off + CHUNK_R

    pltpu.async_copy(zeros_hbm.at[:], zbuf, szi).wait()
    zh = [pltpu.async_copy(
        zbuf, out_hbm.at[pl.ds(zero_off + j * ZBR, ZBR)], sz)
        for j in range(ZCH)]

    in_h = [None] * NCH
    out_h = [None] * NCH
    for j in range(NCH):
        b = j % NBUF
        if j >= NBUF:
            out_h[j - NBUF].wait()
        in_h[j] = pltpu.async_copy(
            in_hbm.at[pl.ds(in_off + j * BR, BR)], bufs[b], sin[b])
        in_h[j].wait()
        out_h[j] = pltpu.async_copy(
            bufs[b], out_hbm.at[pl.ds(out_off + j * BR, BR)], sout[b])
    for j in range(max(0, NCH - NBUF), NCH):
        out_h[j].wait()
    for h in zh:
        h.wait()


def kernel(psi):
    flat = psi.reshape(ROWS // LANE, LANE, COLS).transpose(0, 2, 1)
    flat = flat.reshape(IN_R, LANE)
    zeros = jnp.zeros((ZBR, LANE), jnp.float32)
    mesh = plsc.VectorSubcoreMesh(core_axis_name="c", subcore_axis_name="s")
    run = pl.kernel(
        _body,
        out_type=jax.ShapeDtypeStruct((OUT_R, LANE), jnp.float32),
        mesh=mesh,
        scratch_types=(
            [pltpu.VMEM((BR, LANE), jnp.float32)] * NBUF
            + [pltpu.VMEM((ZBR, LANE), jnp.float32)]
            + [pltpu.SemaphoreType.DMA] * 8
        ),
    )
    out = run(flat, zeros)
    out = out.reshape(2 * ROWS // LANE, COLS, LANE).transpose(0, 2, 1)
    return out.reshape(2 * ROWS, COLS)
